# baseline (device time: 182459 ns/iter reference)
import functools

import jax
import jax.numpy as jnp
from jax import lax
from jax.experimental import pallas as pl
from jax.experimental.pallas import tpu as pltpu

P = 16


def kernel(x, w_mat):
    m, k = x.shape
    k2, n = w_mat.shape
    nper = n // P

    def body(pos_ref, x_ref, w_ref, out_ref, comm, send_sems, recv_sems):
        s = pl.program_id(0)
        my_pos = pos_ref[0]

        @pl.when(s == 0)
        def _():
            barrier_sem = pltpu.get_barrier_semaphore()
            for p in range(P):
                pl.semaphore_signal(
                    barrier_sem, inc=1,
                    device_id=(p,), device_id_type=pl.DeviceIdType.MESH,
                )
            pl.semaphore_wait(barrier_sem, P)

        yblk = jnp.dot(x_ref[:, :], w_ref[:, :],
                       preferred_element_type=jnp.float32)
        yblk = yblk * jax.nn.sigmoid(yblk)

        @pl.when(s == 0)
        def _():
            out_ref[pl.ds(my_pos * m, m), :] = yblk

        @pl.when(s > 0)
        def _():
            dst = lax.rem(my_pos + s, P)
            slot = lax.rem(s, 2)
            comm[slot] = yblk
            rdma = pltpu.make_async_remote_copy(
                src_ref=comm.at[slot],
                dst_ref=out_ref.at[pl.ds(my_pos * m, m), :],
                send_sem=send_sems.at[slot],
                recv_sem=recv_sems.at[s],
                device_id=(dst,),
                device_id_type=pl.DeviceIdType.MESH,
            )
            rdma.start()
            rdma.wait_send()

        @pl.when(s == P - 1)
        def _():
            for t in range(1, P):
                src = lax.rem(my_pos - t + P, P)
                recv = pltpu.make_async_remote_copy(
                    src_ref=comm.at[0],
                    dst_ref=out_ref.at[pl.ds(src * m, m), :],
                    send_sem=send_sems.at[0],
                    recv_sem=recv_sems.at[t],
                    device_id=(src,),
                    device_id_type=pl.DeviceIdType.MESH,
                )
                recv.wait_recv()

    grid_spec = pltpu.PrefetchScalarGridSpec(
        num_scalar_prefetch=1,
        grid=(P,),
        in_specs=[
            pl.BlockSpec((m, k), lambda s, pos: (0, 0)),
            pl.BlockSpec((k, nper), lambda s, pos: (0, lax.rem(pos[0] + s, P))),
        ],
        out_specs=pl.BlockSpec((P * m, nper), lambda s, pos: (0, 0)),
        scratch_shapes=[
            pltpu.VMEM((2, m, nper), jnp.float32),
            pltpu.SemaphoreType.DMA((2,)),
            pltpu.SemaphoreType.DMA((P,)),
        ],
    )

    my_pos = jnp.array([lax.axis_index("i")], dtype=jnp.int32)
    return pl.pallas_call(
        body,
        grid_spec=grid_spec,
        out_shape=jax.ShapeDtypeStruct((P * m, nper), jnp.float32),
        compiler_params=pltpu.CompilerParams(collective_id=0),
    )(my_pos, x, w_mat)


# device time: 139323 ns/iter; 1.3096x vs baseline; 1.3096x over previous
import jax
import jax.numpy as jnp
from jax import lax
from jax.experimental import pallas as pl
from jax.experimental.pallas import tpu as pltpu

P = 16
NSLOT = 4

PERM = [8, 7, 9, 6, 10, 5, 11, 4, 12, 3, 13, 2, 14, 1, 15, 0]


def kernel(x, w_mat):
    m, k = x.shape
    k2, n = w_mat.shape
    nper = n // P

    def body(pos_ref, x_ref, w_ref, out_ref, comm, send_sems, recv_sems):
        s = pl.program_id(0)
        my_pos = pos_ref[0]
        my_rows = pl.ds(my_pos * m, m)

        @pl.when(s == 0)
        def _():
            barrier_sem = pltpu.get_barrier_semaphore()
            for p in range(P):
                pl.semaphore_signal(
                    barrier_sem, inc=1,
                    device_id=(p,), device_id_type=pl.DeviceIdType.MESH,
                )
            pl.semaphore_wait(barrier_sem, P)

        @pl.when(jnp.logical_and(s >= NSLOT, s <= P - 2))
        def _():
            slot = lax.rem(s, NSLOT)
            pltpu.make_async_remote_copy(
                src_ref=comm.at[slot],
                dst_ref=comm.at[slot],
                send_sem=send_sems.at[slot],
                recv_sem=recv_sems.at[P - 1],
                device_id=(0,),
                device_id_type=pl.DeviceIdType.MESH,
            ).wait_send()

        yblk = jnp.dot(x_ref[:, :], w_ref[:, :],
                       preferred_element_type=jnp.float32)
        yblk = yblk * jax.nn.sigmoid(yblk)

        @pl.when(s <= P - 2)
        def _():
            dst = lax.rem(my_pos + pos_ref[s + 1], P)
            slot = lax.rem(s, NSLOT)
            comm[slot] = yblk
            pltpu.make_async_remote_copy(
                src_ref=comm.at[slot],
                dst_ref=out_ref.at[my_rows, :],
                send_sem=send_sems.at[slot],
                recv_sem=recv_sems.at[s],
                device_id=(dst,),
                device_id_type=pl.DeviceIdType.MESH,
            ).start()

        @pl.when(s == P - 1)
        def _():
            out_ref[my_rows, :] = yblk

            for t in range(P - 1):
                src = lax.rem(my_pos - PERM[t] + 2 * P, P)
                pltpu.make_async_remote_copy(
                    src_ref=comm.at[0],
                    dst_ref=out_ref.at[pl.ds(src * m, m), :],
                    send_sem=send_sems.at[0],
                    recv_sem=recv_sems.at[t],
                    device_id=(src,),
                    device_id_type=pl.DeviceIdType.MESH,
                ).wait_recv()

            for t in range(P - 1 - NSLOT, P - 1):
                pltpu.make_async_remote_copy(
                    src_ref=comm.at[t % NSLOT],
                    dst_ref=comm.at[t % NSLOT],
                    send_sem=send_sems.at[t % NSLOT],
                    recv_sem=recv_sems.at[P - 1],
                    device_id=(0,),
                    device_id_type=pl.DeviceIdType.MESH,
                ).wait_send()

    grid_spec = pltpu.PrefetchScalarGridSpec(
        num_scalar_prefetch=1,
        grid=(P,),
        in_specs=[
            pl.BlockSpec((m, k), lambda s, pos: (0, 0)),
            pl.BlockSpec(
                (k, nper),
                lambda s, pos: (0, lax.rem(pos[0] + pos[s + 1], P)),
            ),
        ],
        out_specs=pl.BlockSpec((P * m, nper), lambda s, pos: (0, 0)),
        scratch_shapes=[
            pltpu.VMEM((NSLOT, m, nper), jnp.float32),
            pltpu.SemaphoreType.DMA((NSLOT,)),
            pltpu.SemaphoreType.DMA((P,)),
        ],
    )

    scalars = jnp.concatenate([
        jnp.array([lax.axis_index("i")], dtype=jnp.int32),
        jnp.array(PERM, dtype=jnp.int32),
    ])
    return pl.pallas_call(
        body,
        grid_spec=grid_spec,
        out_shape=jax.ShapeDtypeStruct((P * m, nper), jnp.float32),
        compiler_params=pltpu.CompilerParams(collective_id=0),
    )(scalars, x, w_mat)
